# Initial kernel scaffold; baseline (speedup 1.0000x reference)
import functools

import jax
import jax.numpy as jnp
from jax import lax
from jax.experimental import pallas as pl
from jax.experimental.pallas import tpu as pltpu
from jax.experimental.pallas import tpu_sc as plsc

L = 16
NUM_SPECIES = 16


@functools.cache
def _build(n):
    info = plsc.get_sparse_core_info()
    nw = info.num_cores * info.num_subcores
    assert n % L == 0
    chunk = ((n + nw - 1) // nw + L - 1) // L * L
    tail = n - (nw - 1) * chunk
    assert 0 < tail <= chunk and tail % L == 0

    mesh = plsc.VectorSubcoreMesh(core_axis_name="c", subcore_axis_name="s")

    @functools.partial(
        pl.kernel,
        mesh=mesh,
        out_type=jax.ShapeDtypeStruct((n,), jnp.float32),
        scratch_types=[
            pltpu.VMEM((chunk,), jnp.int32),
            pltpu.VMEM((chunk,), jnp.float32),
            pltpu.VMEM((chunk,), jnp.float32),
            pltpu.VMEM((NUM_SPECIES,), jnp.float32),
            pltpu.VMEM((NUM_SPECIES,), jnp.float32),
        ],
    )
    def sc_kernel(x_hbm, t_hbm, shift_hbm, scale_hbm, out_hbm,
                  idx_v, x_v, out_v, shift_v, scale_v):
        wid = lax.axis_index("s") * info.num_cores + lax.axis_index("c")
        base = wid * chunk

        pltpu.sync_copy(shift_hbm, shift_v)
        pltpu.sync_copy(scale_hbm, scale_v)

        def run(c):
            pltpu.sync_copy(t_hbm.at[pl.ds(base, c)], idx_v.at[pl.ds(0, c)])
            pltpu.sync_copy(x_hbm.at[pl.ds(base, c)], x_v.at[pl.ds(0, c)])

            def body(i, carry):
                o = i * L
                idx = idx_v[pl.ds(o, L)]
                xv = x_v[pl.ds(o, L)]
                s = plsc.load_gather(scale_v, [idx])
                b = plsc.load_gather(shift_v, [idx])
                out_v[pl.ds(o, L)] = xv * s + b
                return carry

            lax.fori_loop(0, c // L, body, 0)
            pltpu.sync_copy(out_v.at[pl.ds(0, c)], out_hbm.at[pl.ds(base, c)])

        @pl.when(wid < nw - 1)
        def _():
            run(chunk)

        @pl.when(wid == nw - 1)
        def _():
            run(tail)

    return sc_kernel


def kernel(scaled_atomic_energy, atom_type, shift, scale):
    n = scaled_atomic_energy.shape[0]
    x = scaled_atomic_energy.reshape(-1)
    t = atom_type.astype(jnp.int32)
    out = _build(n)(x, t, shift.astype(jnp.float32), scale.astype(jnp.float32))
    return out.reshape(n, 1)


# speedup vs baseline: 1.0598x; 1.0598x over previous
import functools

import jax
import jax.numpy as jnp
from jax import lax
from jax.experimental import pallas as pl
from jax.experimental.pallas import tpu as pltpu
from jax.experimental.pallas import tpu_sc as plsc

L = 16
NUM_SPECIES = 16

_GATHER_DNUMS = lax.GatherDimensionNumbers(
    offset_dims=(), collapsed_slice_dims=(0,), start_index_map=(0,))


def _vreg_gather(tab, idx):
    return lax.gather(
        tab, idx[:, None], _GATHER_DNUMS, slice_sizes=(1,),
        mode=lax.GatherScatterMode.PROMISE_IN_BOUNDS)


@functools.cache
def _build(n):
    info = plsc.get_sparse_core_info()
    nw = info.num_cores * info.num_subcores
    assert n % L == 0
    chunk = ((n + nw - 1) // nw + L - 1) // L * L
    tail = n - (nw - 1) * chunk
    assert 0 < tail <= chunk and tail % L == 0

    mesh = plsc.VectorSubcoreMesh(core_axis_name="c", subcore_axis_name="s")

    @functools.partial(
        pl.kernel,
        mesh=mesh,
        out_type=jax.ShapeDtypeStruct((n,), jnp.float32),
        scratch_types=[
            pltpu.VMEM((chunk,), jnp.int32),
            pltpu.VMEM((chunk,), jnp.float32),
            pltpu.VMEM((chunk,), jnp.float32),
            pltpu.VMEM((NUM_SPECIES,), jnp.float32),
            pltpu.VMEM((NUM_SPECIES,), jnp.float32),
        ],
    )
    def sc_kernel(x_hbm, t_hbm, shift_hbm, scale_hbm, out_hbm,
                  idx_v, x_v, out_v, shift_v, scale_v):
        wid = lax.axis_index("s") * info.num_cores + lax.axis_index("c")
        base = wid * chunk

        pltpu.sync_copy(shift_hbm, shift_v)
        pltpu.sync_copy(scale_hbm, scale_v)

        def run(c):
            pltpu.sync_copy(t_hbm.at[pl.ds(base, c)], idx_v.at[pl.ds(0, c)])
            pltpu.sync_copy(x_hbm.at[pl.ds(base, c)], x_v.at[pl.ds(0, c)])
            s_tab = scale_v[...]
            b_tab = shift_v[...]

            def body(i, carry):
                o = i * L
                idx = idx_v[pl.ds(o, L)]
                xv = x_v[pl.ds(o, L)]
                s = _vreg_gather(s_tab, idx)
                b = _vreg_gather(b_tab, idx)
                out_v[pl.ds(o, L)] = xv * s + b
                return carry

            lax.fori_loop(0, c // L, body, 0)
            pltpu.sync_copy(out_v.at[pl.ds(0, c)], out_hbm.at[pl.ds(base, c)])

        @pl.when(wid < nw - 1)
        def _():
            run(chunk)

        @pl.when(wid == nw - 1)
        def _():
            run(tail)

    return sc_kernel


def kernel(scaled_atomic_energy, atom_type, shift, scale):
    n = scaled_atomic_energy.shape[0]
    x = scaled_atomic_energy.reshape(-1)
    t = atom_type.astype(jnp.int32)
    out = _build(n)(x, t, shift.astype(jnp.float32), scale.astype(jnp.float32))
    return out.reshape(n, 1)


# speedup vs baseline: 1.1523x; 1.0873x over previous
import functools

import jax
import jax.numpy as jnp
from jax import lax
from jax.experimental import pallas as pl
from jax.experimental.pallas import tpu as pltpu
from jax.experimental.pallas import tpu_sc as plsc

L = 16
NUM_SPECIES = 16

_GATHER_DNUMS = lax.GatherDimensionNumbers(
    offset_dims=(), collapsed_slice_dims=(0,), start_index_map=(0,))


def _vreg_gather(tab, idx):
    return lax.gather(
        tab, idx[:, None], _GATHER_DNUMS, slice_sizes=(1,),
        mode=lax.GatherScatterMode.PROMISE_IN_BOUNDS)


@functools.cache
def _build(n):
    info = plsc.get_sparse_core_info()
    nw = info.num_cores * info.num_subcores
    assert n % L == 0
    chunk = ((n + nw - 1) // nw + L - 1) // L * L
    tail = n - (nw - 1) * chunk
    assert 0 < tail <= chunk and tail % L == 0

    mesh = plsc.VectorSubcoreMesh(core_axis_name="c", subcore_axis_name="s")

    @functools.partial(
        pl.kernel,
        mesh=mesh,
        out_type=jax.ShapeDtypeStruct((n,), jnp.float32),
        scratch_types=[
            pltpu.VMEM((chunk,), jnp.int32),
            pltpu.VMEM((chunk,), jnp.float32),
            pltpu.VMEM((chunk,), jnp.float32),
            pltpu.VMEM((NUM_SPECIES,), jnp.float32),
            pltpu.VMEM((NUM_SPECIES,), jnp.float32),
            pltpu.SemaphoreType.DMA,
        ],
    )
    def sc_kernel(x_hbm, t_hbm, shift_hbm, scale_hbm, out_hbm,
                  idx_v, x_v, out_v, shift_v, scale_v, sem):
        wid = lax.axis_index("s") * info.num_cores + lax.axis_index("c")
        base = wid * chunk

        def run(c):
            cps = [
                pltpu.async_copy(shift_hbm, shift_v, sem),
                pltpu.async_copy(scale_hbm, scale_v, sem),
                pltpu.async_copy(t_hbm.at[pl.ds(base, c)],
                                 idx_v.at[pl.ds(0, c)], sem),
                pltpu.async_copy(x_hbm.at[pl.ds(base, c)],
                                 x_v.at[pl.ds(0, c)], sem),
            ]
            for cp in cps:
                cp.wait()
            s_tab = scale_v[...]
            b_tab = shift_v[...]

            @plsc.parallel_loop(0, c // L, unroll=4)
            def body(i):
                o = i * L
                idx = idx_v[pl.ds(o, L)]
                xv = x_v[pl.ds(o, L)]
                s = _vreg_gather(s_tab, idx)
                b = _vreg_gather(b_tab, idx)
                out_v[pl.ds(o, L)] = xv * s + b

            pltpu.sync_copy(out_v.at[pl.ds(0, c)], out_hbm.at[pl.ds(base, c)])

        @pl.when(wid < nw - 1)
        def _():
            run(chunk)

        @pl.when(wid == nw - 1)
        def _():
            run(tail)

    return sc_kernel


def kernel(scaled_atomic_energy, atom_type, shift, scale):
    n = scaled_atomic_energy.shape[0]
    x = scaled_atomic_energy.reshape(-1)
    t = atom_type.astype(jnp.int32)
    out = _build(n)(x, t, shift.astype(jnp.float32), scale.astype(jnp.float32))
    return out.reshape(n, 1)


# speedup vs baseline: 1.1524x; 1.0001x over previous
import functools

import jax
import jax.numpy as jnp
from jax import lax
from jax.experimental import pallas as pl
from jax.experimental.pallas import tpu as pltpu
from jax.experimental.pallas import tpu_sc as plsc

L = 16
NUM_SPECIES = 16

_GATHER_DNUMS = lax.GatherDimensionNumbers(
    offset_dims=(), collapsed_slice_dims=(0,), start_index_map=(0,))


def _vreg_gather(tab, idx):
    return lax.gather(
        tab, idx[:, None], _GATHER_DNUMS, slice_sizes=(1,),
        mode=lax.GatherScatterMode.PROMISE_IN_BOUNDS)


@functools.cache
def _build(n):
    info = plsc.get_sparse_core_info()
    nw = info.num_cores * info.num_subcores
    assert n % L == 0
    chunk = ((n + nw - 1) // nw + L - 1) // L * L
    chunk = (chunk + 2 * L - 1) // (2 * L) * (2 * L)
    assert chunk * (nw - 1) + chunk >= n and n - chunk >= 0
    half = chunk // 2

    mesh = plsc.VectorSubcoreMesh(core_axis_name="c", subcore_axis_name="s")

    @functools.partial(
        pl.kernel,
        mesh=mesh,
        out_type=jax.ShapeDtypeStruct((n,), jnp.float32),
        scratch_types=[
            pltpu.VMEM((chunk,), jnp.int32),
            pltpu.VMEM((chunk,), jnp.float32),
            pltpu.VMEM((chunk,), jnp.float32),
            pltpu.VMEM((NUM_SPECIES,), jnp.float32),
            pltpu.VMEM((NUM_SPECIES,), jnp.float32),
            pltpu.SemaphoreType.DMA,
            pltpu.SemaphoreType.DMA,
            pltpu.SemaphoreType.DMA,
            pltpu.SemaphoreType.DMA,
        ],
    )
    def sc_kernel(x_hbm, t_hbm, shift_hbm, scale_hbm, out_hbm,
                  idx_v, x_v, out_v, shift_v, scale_v,
                  sem_t, sem_a, sem_b, sem_o):
        wid = lax.axis_index("s") * info.num_cores + lax.axis_index("c")
        base = jnp.minimum(wid * chunk, n - chunk)

        cps_t = [pltpu.async_copy(shift_hbm, shift_v, sem_t),
                 pltpu.async_copy(scale_hbm, scale_v, sem_t)]
        cps_a = [pltpu.async_copy(t_hbm.at[pl.ds(base, half)],
                                  idx_v.at[pl.ds(0, half)], sem_a),
                 pltpu.async_copy(x_hbm.at[pl.ds(base, half)],
                                  x_v.at[pl.ds(0, half)], sem_a)]
        cps_b = [pltpu.async_copy(t_hbm.at[pl.ds(base + half, half)],
                                  idx_v.at[pl.ds(half, half)], sem_b),
                 pltpu.async_copy(x_hbm.at[pl.ds(base + half, half)],
                                  x_v.at[pl.ds(half, half)], sem_b)]

        for cp in cps_t:
            cp.wait()
        s_tab = scale_v[...]
        b_tab = shift_v[...]

        def compute(lo):
            @plsc.parallel_loop(lo // L, (lo + half) // L, unroll=4)
            def body(i):
                o = i * L
                idx = idx_v[pl.ds(o, L)]
                xv = x_v[pl.ds(o, L)]
                s = _vreg_gather(s_tab, idx)
                b = _vreg_gather(b_tab, idx)
                out_v[pl.ds(o, L)] = xv * s + b

        for cp in cps_a:
            cp.wait()
        compute(0)
        out_a = pltpu.async_copy(out_v.at[pl.ds(0, half)],
                                 out_hbm.at[pl.ds(base, half)], sem_o)
        for cp in cps_b:
            cp.wait()
        compute(half)
        out_b = pltpu.async_copy(out_v.at[pl.ds(half, half)],
                                 out_hbm.at[pl.ds(base + half, half)], sem_o)
        out_a.wait()
        out_b.wait()

    return sc_kernel


def kernel(scaled_atomic_energy, atom_type, shift, scale):
    n = scaled_atomic_energy.shape[0]
    x = scaled_atomic_energy.reshape(-1)
    t = atom_type.astype(jnp.int32)
    out = _build(n)(x, t, shift.astype(jnp.float32), scale.astype(jnp.float32))
    return out.reshape(n, 1)
